# baseline (speedup 1.0000x reference)
import functools

import jax
import jax.numpy as jnp
from jax import lax
from jax.experimental import pallas as pl
from jax.experimental.pallas import tpu as pltpu
from jax.experimental.pallas import tpu_sc as plsc

_NC = 2
_NS = 16
_NW = _NC * _NS
_LANES = 16


@functools.lru_cache(maxsize=None)
def _build_plane_gather(L: int, B: int, V: int, D: int):
    d_per_w = D // _NW
    n_vec = B // _LANES
    mesh = plsc.VectorSubcoreMesh(core_axis_name="c", subcore_axis_name="s")

    @functools.partial(
        pl.kernel,
        out_type=jax.ShapeDtypeStruct((L, D, B), jnp.float32),
        mesh=mesh,
        scratch_types=[
            pltpu.VMEM_SHARED((L, B), jnp.int32),
            pltpu.VMEM((V,), jnp.float32),
            pltpu.VMEM((B,), jnp.int32),
            [pltpu.VMEM((B,), jnp.float32) for _ in range(2)],
            [pltpu.SemaphoreType.DMA for _ in range(2)],
        ],
        compiler_params=pltpu.CompilerParams(
            use_tc_tiling_on_sc=False, needs_layout_passes=False
        ),
    )
    def plane_kernel(tableT_hbm, idxT_hbm, out_hbm, idx_sh, trow_v, idx_v,
                     plane, wsem):
        sid = lax.axis_index("s")
        wid = sid * _NC + lax.axis_index("c")

        @pl.when(sid == 0)
        def _():
            pltpu.sync_copy(idxT_hbm, idx_sh)

        plsc.subcore_barrier()

        for dd in range(d_per_w):
            d = wid * d_per_w + dd
            pltpu.sync_copy(tableT_hbm.at[d], trow_v)

            def pair(g, carry, d=d, dd=dd):
                for p in range(2):
                    l = g * 2 + p
                    pltpu.sync_copy(idx_sh.at[l], idx_v)

                    def drain(p=p):
                        pltpu.make_async_copy(
                            plane[p], out_hbm.at[0, 0], wsem[p]
                        ).wait()

                    if dd == 0:
                        pl.when(g > 0)(drain)
                    else:
                        drain()

                    def vec(k, carry2, p=p):
                        iv = idx_v[pl.ds(k * _LANES, _LANES)]
                        plane[p][pl.ds(k * _LANES, _LANES)] = plsc.load_gather(
                            trow_v, [iv]
                        )
                        return carry2

                    lax.fori_loop(0, n_vec, vec, 0, unroll=8)
                    pltpu.async_copy(plane[p], out_hbm.at[l, d], wsem[p])
                return carry

            lax.fori_loop(0, L // 2, pair, 0)

        for p in range(2):
            pltpu.make_async_copy(plane[p], out_hbm.at[0, 0], wsem[p]).wait()

    return plane_kernel


def kernel(sentences, lengths, bert_sent, bert_sent_type, bert_sent_mask, embed_table):
    B, L = sentences.shape
    V, D = embed_table.shape
    out = _build_plane_gather(L, B, V, D)(
        embed_table.T, sentences.T.astype(jnp.int32)
    )
    return out.transpose(2, 0, 1)


# speedup vs baseline: 2.3748x; 2.3748x over previous
import functools

import jax
import jax.numpy as jnp
from jax import lax
from jax.experimental import pallas as pl
from jax.experimental.pallas import tpu as pltpu
from jax.experimental.pallas import tpu_sc as plsc

_NC = 2
_NS = 16
_NW = _NC * _NS
_LANES = 16


@functools.lru_cache(maxsize=None)
def _build_plane_gather(L: int, B: int, V: int, D: int):
    d_per_w = D // _NW
    mesh = plsc.VectorSubcoreMesh(core_axis_name="c", subcore_axis_name="s")

    @functools.partial(
        pl.kernel,
        out_type=jax.ShapeDtypeStruct((L, D, B), jnp.float32),
        mesh=mesh,
        scratch_types=[
            pltpu.VMEM_SHARED((L, B), jnp.int32),
            pltpu.VMEM((V,), jnp.float32),
            [pltpu.VMEM((B,), jnp.int32) for _ in range(2)],
            [pltpu.VMEM((B,), jnp.float32) for _ in range(2)],
            [pltpu.SemaphoreType.DMA for _ in range(2)],
            [pltpu.SemaphoreType.DMA for _ in range(2)],
        ],
        compiler_params=pltpu.CompilerParams(
            use_tc_tiling_on_sc=False, needs_layout_passes=False
        ),
    )
    def plane_kernel(tableT_hbm, idxT_hbm, out_hbm, idx_sh, trow_v, idx_v,
                     plane, isem, wsem):
        sid = lax.axis_index("s")
        wid = sid * _NC + lax.axis_index("c")

        @pl.when(sid == 0)
        def _():
            pltpu.sync_copy(idxT_hbm, idx_sh)

        plsc.subcore_barrier()

        pltpu.async_copy(idx_sh.at[0], idx_v[0], isem[0])

        for dd in range(d_per_w):
            d = wid * d_per_w + dd
            pltpu.sync_copy(tableT_hbm.at[d], trow_v)

            def pair(g, carry, d=d, dd=dd):
                for p in range(2):
                    l = g * 2 + p
                    pltpu.make_async_copy(
                        idx_sh.at[0], idx_v[p], isem[p]
                    ).wait()
                    l_next = lax.rem(l + 1, L)
                    pltpu.async_copy(
                        idx_sh.at[l_next], idx_v[1 - p], isem[1 - p]
                    )

                    def drain(p=p):
                        pltpu.make_async_copy(
                            plane[p], out_hbm.at[0, 0], wsem[p]
                        ).wait()

                    if dd == 0:
                        pl.when(g > 0)(drain)
                    else:
                        drain()

                    @plsc.parallel_loop(0, B, _LANES, unroll=8)
                    def _(i, p=p):
                        iv = idx_v[p][pl.ds(i, _LANES)]
                        plane[p][pl.ds(i, _LANES)] = plsc.load_gather(
                            trow_v, [iv]
                        )

                    pltpu.async_copy(plane[p], out_hbm.at[l, d], wsem[p])
                return carry

            lax.fori_loop(0, L // 2, pair, 0)

        pltpu.make_async_copy(idx_sh.at[0], idx_v[0], isem[0]).wait()
        for p in range(2):
            pltpu.make_async_copy(plane[p], out_hbm.at[0, 0], wsem[p]).wait()

    return plane_kernel


def kernel(sentences, lengths, bert_sent, bert_sent_type, bert_sent_mask, embed_table):
    B, L = sentences.shape
    V, D = embed_table.shape
    out = _build_plane_gather(L, B, V, D)(
        embed_table.T, sentences.T.astype(jnp.int32)
    )
    return out.transpose(2, 0, 1)


# speedup vs baseline: 3.5831x; 1.5088x over previous
import functools

import jax
import jax.numpy as jnp
from jax import lax
from jax.experimental import pallas as pl
from jax.experimental.pallas import tpu as pltpu
from jax.experimental.pallas import tpu_sc as plsc

_NC = 2
_NS = 16
_NW = _NC * _NS
_LANES = 16


@functools.lru_cache(maxsize=None)
def _build_plane_gather(L: int, B: int, V: int, D: int):
    d_per_w = D // _NW
    tr_n = D // 8
    tc_n = B // 128
    mesh = plsc.VectorSubcoreMesh(core_axis_name="c", subcore_axis_name="s")

    @functools.partial(
        pl.kernel,
        out_type=jax.ShapeDtypeStruct((L, tr_n, tc_n, 8, 128), jnp.float32),
        mesh=mesh,
        scratch_types=[
            pltpu.VMEM_SHARED((L, B), jnp.int32),
            pltpu.VMEM((V,), jnp.float32),
            [pltpu.VMEM((B,), jnp.int32) for _ in range(2)],
            [pltpu.VMEM((tc_n, 128), jnp.float32) for _ in range(2)],
            [pltpu.SemaphoreType.DMA for _ in range(2)],
            [pltpu.SemaphoreType.DMA for _ in range(2)],
        ],
        compiler_params=pltpu.CompilerParams(
            use_tc_tiling_on_sc=False, needs_layout_passes=False
        ),
    )
    def plane_kernel(tableT_hbm, idxT_hbm, out_hbm, idx_sh, trow_v, idx_v,
                     plane, isem, wsem):
        sid = lax.axis_index("s")
        wid = sid * _NC + lax.axis_index("c")

        @pl.when(sid == 0)
        def _():
            pltpu.sync_copy(idxT_hbm, idx_sh)

        plsc.subcore_barrier()

        pltpu.async_copy(idx_sh.at[0], idx_v[0], isem[0])

        for dd in range(d_per_w):
            d = wid * d_per_w + dd
            pltpu.sync_copy(tableT_hbm.at[d], trow_v)

            def pair(g, carry, d=d, dd=dd):
                for p in range(2):
                    l = g * 2 + p
                    pltpu.make_async_copy(
                        idx_sh.at[0], idx_v[p], isem[p]
                    ).wait()
                    l_next = lax.rem(l + 1, L)
                    pltpu.async_copy(
                        idx_sh.at[l_next], idx_v[1 - p], isem[1 - p]
                    )

                    def drain(p=p):
                        pltpu.make_async_copy(
                            plane[p], out_hbm.at[0, 0, :, 0, :], wsem[p]
                        ).wait()

                    if dd == 0:
                        pl.when(g > 0)(drain)
                    else:
                        drain()

                    @plsc.parallel_loop(0, B, _LANES, unroll=8)
                    def _(i, p=p):
                        iv = idx_v[p][pl.ds(i, _LANES)]
                        plane[p][i // 128, pl.ds(lax.rem(i, 128), _LANES)] = (
                            plsc.load_gather(trow_v, [iv])
                        )

                    pltpu.async_copy(
                        plane[p], out_hbm.at[l, d // 8, :, lax.rem(d, 8), :],
                        wsem[p],
                    )
                return carry

            lax.fori_loop(0, L // 2, pair, 0)

        pltpu.make_async_copy(idx_sh.at[0], idx_v[0], isem[0]).wait()
        for p in range(2):
            pltpu.make_async_copy(
                plane[p], out_hbm.at[0, 0, :, 0, :], wsem[p]
            ).wait()

    return plane_kernel


def kernel(sentences, lengths, bert_sent, bert_sent_type, bert_sent_mask, embed_table):
    B, L = sentences.shape
    V, D = embed_table.shape
    out = _build_plane_gather(L, B, V, D)(
        embed_table.T, sentences.T.astype(jnp.int32)
    )
    return out.transpose(2, 4, 0, 1, 3).reshape(B, L, D)


# speedup vs baseline: 5.4527x; 1.5218x over previous
import functools

import jax
import jax.numpy as jnp
from jax import lax
from jax.experimental import pallas as pl
from jax.experimental.pallas import tpu as pltpu
from jax.experimental.pallas import tpu_sc as plsc

_NC = 2
_NS = 16
_NW = _NC * _NS
_LANES = 16


@functools.lru_cache(maxsize=None)
def _build_plane_gather(L: int, B: int, V: int, D: int):
    d_per_w = D // _NW
    tr_n = D // 8
    mesh = plsc.VectorSubcoreMesh(core_axis_name="c", subcore_axis_name="s")

    @functools.partial(
        pl.kernel,
        out_type=jax.ShapeDtypeStruct((L, tr_n, 8, B), jnp.float32),
        mesh=mesh,
        scratch_types=[
            pltpu.VMEM_SHARED((L * B,), jnp.int32),
            pltpu.VMEM((V,), jnp.float32),
            [pltpu.VMEM((B,), jnp.int32) for _ in range(2)],
            [pltpu.VMEM((B,), jnp.float32) for _ in range(2)],
            [pltpu.SemaphoreType.DMA for _ in range(2)],
            [pltpu.SemaphoreType.DMA for _ in range(2)],
        ],
        compiler_params=pltpu.CompilerParams(
            use_tc_tiling_on_sc=True, needs_layout_passes=False
        ),
    )
    def plane_kernel(tableT_hbm, idx_hbm, out_hbm, idx_sh, trow_v, idx_v,
                     plane, isem, wsem):
        sid = lax.axis_index("s")
        wid = sid * _NC + lax.axis_index("c")

        @pl.when(sid == 0)
        def _():
            pltpu.sync_copy(idx_hbm, idx_sh)

        plsc.subcore_barrier()

        pltpu.async_copy(idx_sh.at[pl.ds(0, B)], idx_v[0], isem[0])

        for dd in range(d_per_w):
            d = wid * d_per_w + dd
            pltpu.sync_copy(tableT_hbm.at[d], trow_v)

            def pair(g, carry, d=d, dd=dd):
                for p in range(2):
                    l = g * 2 + p
                    pltpu.make_async_copy(
                        idx_sh.at[pl.ds(0, B)], idx_v[p], isem[p]
                    ).wait()
                    l_next = lax.rem(l + 1, L)
                    pltpu.async_copy(
                        idx_sh.at[pl.ds(l_next * B, B)], idx_v[1 - p],
                        isem[1 - p],
                    )

                    def drain(p=p):
                        pltpu.make_async_copy(
                            plane[p], out_hbm.at[0, 0, 0], wsem[p]
                        ).wait()

                    if dd == 0:
                        pl.when(g > 0)(drain)
                    else:
                        drain()

                    @plsc.parallel_loop(0, B, _LANES, unroll=8)
                    def _(i, p=p):
                        iv = idx_v[p][pl.ds(i, _LANES)]
                        plane[p][pl.ds(i, _LANES)] = plsc.load_gather(
                            trow_v, [iv]
                        )

                    pltpu.async_copy(
                        plane[p], out_hbm.at[l, d // 8, lax.rem(d, 8)],
                        wsem[p],
                    )
                return carry

            lax.fori_loop(0, L // 2, pair, 0)

        pltpu.make_async_copy(idx_sh.at[pl.ds(0, B)], idx_v[0], isem[0]).wait()
        for p in range(2):
            pltpu.make_async_copy(
                plane[p], out_hbm.at[0, 0, 0], wsem[p]
            ).wait()

    return plane_kernel


def kernel(sentences, lengths, bert_sent, bert_sent_type, bert_sent_mask, embed_table):
    B, L = sentences.shape
    V, D = embed_table.shape
    out = _build_plane_gather(L, B, V, D)(
        embed_table.T, sentences.T.astype(jnp.int32).reshape(-1)
    )
    return out.transpose(3, 0, 1, 2).reshape(B, L, D)


# speedup vs baseline: 5.4808x; 1.0051x over previous
import functools

import jax
import jax.numpy as jnp
from jax import lax
from jax.experimental import pallas as pl
from jax.experimental.pallas import tpu as pltpu
from jax.experimental.pallas import tpu_sc as plsc

_NC = 2
_NS = 16
_NW = _NC * _NS
_LANES = 16


@functools.lru_cache(maxsize=None)
def _build_plane_gather(L: int, B: int, V: int, D: int):
    d_per_w = D // _NW
    tr_n = D // 8
    mesh = plsc.VectorSubcoreMesh(core_axis_name="c", subcore_axis_name="s")

    @functools.partial(
        pl.kernel,
        out_type=jax.ShapeDtypeStruct((L, tr_n, 8, B), jnp.float32),
        mesh=mesh,
        scratch_types=[
            pltpu.VMEM_SHARED((L * B,), jnp.int32),
            pltpu.VMEM((V,), jnp.float32),
            [pltpu.VMEM((B,), jnp.int32) for _ in range(2)],
            [pltpu.VMEM((B,), jnp.float32) for _ in range(2)],
            [pltpu.SemaphoreType.DMA for _ in range(2)],
            [pltpu.SemaphoreType.DMA for _ in range(2)],
        ],
        compiler_params=pltpu.CompilerParams(
            use_tc_tiling_on_sc=True, needs_layout_passes=False
        ),
    )
    def plane_kernel(tableT_hbm, idx_hbm, out_hbm, idx_sh, trow_v, idx_v,
                     plane, isem, wsem):
        sid = lax.axis_index("s")
        wid = sid * _NC + lax.axis_index("c")

        @pl.when(sid == 0)
        def _():
            pltpu.sync_copy(idx_hbm, idx_sh)

        plsc.subcore_barrier()

        pltpu.async_copy(idx_sh.at[pl.ds(0, B)], idx_v[0], isem[0])

        for dd in range(d_per_w):
            d = wid * d_per_w + dd
            pltpu.sync_copy(tableT_hbm.at[d], trow_v)

            def pair(g, carry, d=d, dd=dd):
                for p in range(2):
                    l = g * 2 + p
                    pltpu.make_async_copy(
                        idx_sh.at[pl.ds(0, B)], idx_v[p], isem[p]
                    ).wait()
                    l_next = lax.rem(l + 1, L)
                    pltpu.async_copy(
                        idx_sh.at[pl.ds(l_next * B, B)], idx_v[1 - p],
                        isem[1 - p],
                    )

                    def drain(p=p):
                        pltpu.make_async_copy(
                            plane[p], out_hbm.at[0, 0, 0], wsem[p]
                        ).wait()

                    if dd == 0:
                        pl.when(g > 0)(drain)
                    else:
                        drain()

                    @plsc.parallel_loop(0, B, _LANES, unroll=16)
                    def _(i, p=p):
                        iv = idx_v[p][pl.ds(i, _LANES)]
                        plane[p][pl.ds(i, _LANES)] = plsc.load_gather(
                            trow_v, [iv]
                        )

                    pltpu.async_copy(
                        plane[p], out_hbm.at[l, d // 8, lax.rem(d, 8)],
                        wsem[p],
                    )
                return carry

            lax.fori_loop(0, L // 2, pair, 0)

        pltpu.make_async_copy(idx_sh.at[pl.ds(0, B)], idx_v[0], isem[0]).wait()
        for p in range(2):
            pltpu.make_async_copy(
                plane[p], out_hbm.at[0, 0, 0], wsem[p]
            ).wait()

    return plane_kernel


def kernel(sentences, lengths, bert_sent, bert_sent_type, bert_sent_mask, embed_table):
    B, L = sentences.shape
    V, D = embed_table.shape
    out = _build_plane_gather(L, B, V, D)(
        embed_table.T, sentences.T.astype(jnp.int32).reshape(-1)
    )
    return out.transpose(3, 0, 1, 2).reshape(B, L, D)
